# Initial kernel scaffold; baseline (speedup 1.0000x reference)
import jax
import jax.numpy as jnp
from jax.experimental import pallas as pl


def kernel(nodes, edge_indexs, graph_indicators, W1_0, b1_0, W2_0, b2_0, gamma_0, beta_0, W1_1, b1_1, W2_1, b2_1, gamma_1, beta_1, W1_2, b1_2, W2_2, b2_2, gamma_2, beta_2):
    raise NotImplementedError("write your pallas kernel here")


# speedup vs baseline: 6.0742x; 6.0742x over previous
import functools

import jax
import jax.numpy as jnp
from jax import lax
from jax.experimental import pallas as pl
from jax.experimental.pallas import tpu as pltpu
from jax.experimental.pallas import tpu_sc as plsc

_NC = 2
_NS = 16
_LANES = 16


@functools.lru_cache(maxsize=None)
def _make_scatter(n, d, e):
    nw = _NC * _NS
    assert e % nw == 0
    epw = e // nw
    chunk = 128
    full = epw // chunk
    tail = epw % chunk
    assert tail % 8 == 0
    rpt = (n // _NS) // 8 * 8
    rextra = n - _NS * rpt
    assert rextra % 8 == 0
    zrows = 208
    assert rpt % zrows == 0 and rextra <= zrows
    mesh = plsc.VectorSubcoreMesh(core_axis_name="c", subcore_axis_name="s")

    @functools.partial(
        pl.kernel,
        mesh=mesh,
        out_type=jax.ShapeDtypeStruct((_NC, n, d), jnp.float32),
        scratch_types=[
            pltpu.VMEM((zrows, d), jnp.float32),
            pltpu.VMEM((chunk,), jnp.int32),
            pltpu.VMEM((chunk,), jnp.int32),
            pltpu.VMEM((chunk, d), jnp.float32),
            pltpu.VMEM_SHARED((n, d), jnp.float32),
            pltpu.SemaphoreType.DMA,
        ],
    )
    def scatter_kernel(h_hbm, src_hbm, dst_hbm, out_hbm,
                       zbuf, src_v, dst_v, rows_v, acc_sh, sem):
        cid = lax.axis_index("c")
        sid = lax.axis_index("s")
        wid = sid * _NC + cid

        def zstore(t, _):
            r = t // (d // _LANES)
            c16 = (t % (d // _LANES)) * _LANES
            zbuf[r, pl.ds(c16, _LANES)] = jnp.zeros((_LANES,), jnp.float32)
            return _
        lax.fori_loop(0, zrows * (d // _LANES), zstore, 0)
        row0 = sid * rpt
        def zcopy(j, _):
            pltpu.sync_copy(zbuf, acc_sh.at[pl.ds(row0 + j * zrows, zrows)])
            return _
        lax.fori_loop(0, rpt // zrows, zcopy, 0)
        @pl.when(sid == _NS - 1)
        def _():
            pltpu.sync_copy(zbuf.at[pl.ds(0, rextra)],
                            acc_sh.at[pl.ds(_NS * rpt, rextra)])
        plsc.subcore_barrier()

        e0 = wid * epw
        def body(t, _):
            base = e0 + t * chunk
            pltpu.sync_copy(src_hbm.at[pl.ds(base, chunk)], src_v)
            pltpu.sync_copy(dst_hbm.at[pl.ds(base, chunk)], dst_v)
            pltpu.async_copy(h_hbm.at[src_v], rows_v, sem).wait()
            pltpu.sync_copy(rows_v, acc_sh.at[dst_v], add=True)
            return _
        lax.fori_loop(0, full, body, 0)
        if tail:
            base = e0 + full * chunk
            pltpu.sync_copy(src_hbm.at[pl.ds(base, tail)], src_v.at[pl.ds(0, tail)])
            pltpu.sync_copy(dst_hbm.at[pl.ds(base, tail)], dst_v.at[pl.ds(0, tail)])
            pltpu.async_copy(h_hbm.at[src_v.at[pl.ds(0, tail)]],
                             rows_v.at[pl.ds(0, tail)], sem).wait()
            pltpu.sync_copy(rows_v.at[pl.ds(0, tail)],
                            acc_sh.at[dst_v.at[pl.ds(0, tail)]], add=True)
        plsc.subcore_barrier()

        pltpu.sync_copy(acc_sh.at[pl.ds(row0, rpt)], out_hbm.at[cid].at[pl.ds(row0, rpt)])
        @pl.when(sid == _NS - 1)
        def _():
            pltpu.sync_copy(acc_sh.at[pl.ds(_NS * rpt, rextra)],
                            out_hbm.at[cid].at[pl.ds(_NS * rpt, rextra)])

    return scatter_kernel


@functools.lru_cache(maxsize=None)
def _make_dense(n, d_in, d, relu_out):
    def body(h_ref, a0_ref, a1_ref, w1_ref, b1_ref, w2_ref, b2_ref,
             g_ref, bt_ref, o_ref):
        z = h_ref[...] + a0_ref[...] + a1_ref[...]
        z = jnp.dot(z, w1_ref[...], preferred_element_type=jnp.float32) + b1_ref[...]
        z = jnp.maximum(z, 0.0)
        z = jnp.dot(z, w2_ref[...], preferred_element_type=jnp.float32) + b2_ref[...]
        mu = jnp.mean(z, axis=0, keepdims=True)
        var = jnp.mean((z - mu) * (z - mu), axis=0, keepdims=True)
        z = g_ref[...] * (z - mu) * lax.rsqrt(var + 1e-5) + bt_ref[...]
        if relu_out:
            z = jnp.maximum(z, 0.0)
        o_ref[...] = z

    return pl.pallas_call(
        body,
        out_shape=jax.ShapeDtypeStruct((n, d), jnp.float32),
    )


def kernel(nodes, edge_indexs, graph_indicators,
           W1_0, b1_0, W2_0, b2_0, gamma_0, beta_0,
           W1_1, b1_1, W2_1, b2_1, gamma_1, beta_1,
           W1_2, b1_2, W2_2, b2_2, gamma_2, beta_2):
    del graph_indicators
    n, d = nodes.shape
    e = edge_indexs.shape[1]
    src = edge_indexs[0]
    dst = edge_indexs[1]
    params = [
        (W1_0, b1_0, W2_0, b2_0, gamma_0, beta_0),
        (W1_1, b1_1, W2_1, b2_1, gamma_1, beta_1),
        (W1_2, b1_2, W2_2, b2_2, gamma_2, beta_2),
    ]
    scatter = _make_scatter(n, d, e)
    h = nodes
    for layer, (w1, b1, w2, b2, g, bt) in enumerate(params):
        agg = scatter(h, src, dst)
        dense = _make_dense(n, w1.shape[0], d, layer < len(params) - 1)
        h = dense(h, agg[0], agg[1], w1, b1.reshape(1, d), w2, b2.reshape(1, d),
                  g.reshape(1, d), bt.reshape(1, d))
    return h
